# Initial kernel scaffold; baseline (speedup 1.0000x reference)
import jax
import jax.numpy as jnp
from jax.experimental import pallas as pl


def kernel(x, edge_index, W1, a_src1, a_dst1, b1, W2, a_src2, a_dst2, b2, fcW, fcb):
    raise NotImplementedError("write your pallas kernel here")


# speedup vs baseline: 14.5275x; 14.5275x over previous
import functools

import jax
import jax.numpy as jnp
from jax import lax
from jax.experimental import pallas as pl
from jax.experimental.pallas import tpu as pltpu
from jax.experimental.pallas import tpu_sc as plsc

N = 10000
E = 320000
D = 128
HID = 16
HEADS = 8
OUT = 32

NC = 2
NS = 16
LANES = 16

NPAD = 10112
EB = 32
EP = E + N
PW = ((EP + NS * EB - 1) // (NS * EB)) * EB
EPAD = PW * NS

ROWS_PER_TILE = NPAD // NS
SR = 8


def _edge_pass(nseg, dencols, adstride):
    tw = (nseg + 1) * LANES
    sw = nseg * LANES
    acol = sw
    adn = NPAD * adstride
    multihead = nseg > 1
    mesh = plsc.VectorSubcoreMesh(
        core_axis_name="c", subcore_axis_name="s", num_cores=NC, num_subcores=NS
    )

    @functools.partial(
        pl.kernel,
        out_type=(
            jax.ShapeDtypeStruct((NC, NPAD, sw), jnp.float32),
            jax.ShapeDtypeStruct((NC, NS, adn), jnp.float32),
        ),
        mesh=mesh,
        compiler_params=pltpu.CompilerParams(
            needs_layout_passes=False, use_tc_tiling_on_sc=False),
        scratch_types=[
            pltpu.VMEM((adn,), jnp.float32),
            pltpu.VMEM((adn,), jnp.float32),
            pltpu.VMEM((EB,), jnp.int32),
            pltpu.VMEM((EB,), jnp.int32),
            pltpu.VMEM((EB, tw), jnp.float32),
            pltpu.VMEM((EB, sw), jnp.float32),
            pltpu.VMEM((SR, sw), jnp.float32),
            pltpu.VMEM_SHARED((NPAD, sw), jnp.float32),
            pltpu.SemaphoreType.DMA,
        ],
    )
    def kern(t_hbm, ad_hbm, src_hbm, dst_hbm, out_hbm, den_hbm,
             ad_v, den_v, srcb, dstb, rows, sbuf, stage, acc, sem):
        cid = lax.axis_index("c")
        sid = lax.axis_index("s")
        iot = lax.iota(jnp.int32, LANES)
        zero16 = jnp.zeros((LANES,), jnp.float32)
        ebase = sid * PW
        srcoff = cid * NPAD

        pltpu.sync_copy(ad_hbm.at[pl.ds(cid * adn, adn)], ad_v)

        @pl.loop(0, adn // LANES)
        def _zd(i):
            den_v[pl.ds(i * LANES, LANES)] = zero16

        @pl.loop(0, SR)
        def _zr(r):
            for k in range(sw // LANES):
                stage[r, pl.ds(k * LANES, LANES)] = zero16

        row0 = sid * ROWS_PER_TILE

        @pl.loop(0, ROWS_PER_TILE // SR)
        def _zi(k):
            pltpu.sync_copy(stage, acc.at[pl.ds(row0 + k * SR, SR)])

        plsc.subcore_barrier()

        @pl.loop(0, PW // EB)
        def _batch(b):
            off = ebase + b * EB
            pltpu.sync_copy(src_hbm.at[pl.ds(off, EB)], srcb)
            pltpu.sync_copy(dst_hbm.at[pl.ds(off, EB)], dstb)
            for q in range(EB // LANES):
                srcb[pl.ds(q * LANES, LANES)] = (
                    srcb[pl.ds(q * LANES, LANES)] + srcoff)
            pltpu.async_copy(t_hbm.at[srcb], rows, sem).wait()

            @pl.loop(0, EB // LANES)
            def _grp(q):
                dvec = dstb[pl.ds(q * LANES, LANES)]

                @pl.loop(0, LANES)
                def _edge(e16):
                    e = q * LANES + e16
                    didx = dvec.at[jnp.full((LANES,), e16, jnp.int32)].get(
                        mode="promise_in_bounds")
                    dflat = didx * adstride + (iot & (adstride - 1))
                    ad_row = plsc.load_gather(ad_v, [dflat])
                    a = rows[e, pl.ds(acol, LANES)] + ad_row
                    a = jnp.where(a >= 0.0, a, 0.2 * a)
                    w = jnp.exp(a)
                    if multihead:
                        w = jnp.where(iot < dencols, w, 0.0)
                        plsc.addupdate_scatter(den_v, [dflat], w,
                                               mask=iot < dencols)
                        for h in range(nseg):
                            wb = w.at[jnp.full((LANES,), h, jnp.int32)].get(
                                mode="promise_in_bounds")
                            sbuf[e, pl.ds(h * LANES, LANES)] = (
                                rows[e, pl.ds(h * LANES, LANES)] * wb)
                    else:
                        plsc.addupdate_scatter(den_v, [dflat], w,
                                               mask=iot < dencols)
                        sbuf[e, pl.ds(0, LANES)] = rows[e, pl.ds(0, LANES)] * w

            pltpu.sync_copy(sbuf, acc.at[dstb], add=True)

        pltpu.sync_copy(den_v, den_hbm.at[cid, sid])
        plsc.subcore_barrier()

        @pl.loop(0, ROWS_PER_TILE // SR)
        def _wb(k):
            r = row0 + k * SR
            pltpu.sync_copy(acc.at[pl.ds(r, SR)], stage)
            pltpu.sync_copy(stage, out_hbm.at[cid, pl.ds(r, SR)])

    return kern


_edge_pass_l1 = _edge_pass(4, 4, 4)
_edge_pass_l2 = _edge_pass(1, 1, 1)


_RB = 1264


def _mm_body(x_ref, wcat_ref, wd_ref, t_ref, ad_ref):
    xb = x_ref[...]
    t_ref[...] = jnp.dot(xb, wcat_ref[...], preferred_element_type=jnp.float32)
    ad_ref[...] = jnp.dot(xb, wd_ref[...], preferred_element_type=jnp.float32)


def _feature_mm(x, wcat, wd):
    n, d = x.shape
    wout = wcat.shape[1]
    dout = wd.shape[1]
    return pl.pallas_call(
        _mm_body,
        grid=(n // _RB,),
        in_specs=[
            pl.BlockSpec((_RB, d), lambda i: (i, 0)),
            pl.BlockSpec((d, wout), lambda i: (0, 0)),
            pl.BlockSpec((d, dout), lambda i: (0, 0)),
        ],
        out_specs=[
            pl.BlockSpec((_RB, wout), lambda i: (i, 0)),
            pl.BlockSpec((_RB, dout), lambda i: (i, 0)),
        ],
        out_shape=[
            jax.ShapeDtypeStruct((n, wout), jnp.float32),
            jax.ShapeDtypeStruct((n, dout), jnp.float32),
        ],
    )(x, wcat, wd)


def _layer1_combine_body(acc_ref, den_ref, b1_ref, wcat_ref, wd_ref,
                         t2_ref, ad2_ref):
    segs = []
    for c in range(NC):
        s = acc_ref[c]
        dsum = den_ref[c, 0]
        for t in range(1, NS):
            dsum = dsum + den_ref[c, t]
        recip = 1.0 / (dsum + 1e-16)
        for h4 in range(4):
            segs.append(s[:, h4 * 16:(h4 + 1) * 16] * recip[:, h4:h4 + 1])
    h1 = jnp.concatenate(segs, axis=1) + b1_ref[...]
    h1 = jnp.where(h1 > 0.0, h1, jnp.exp(jnp.minimum(h1, 0.0)) - 1.0)
    t2_ref[...] = jnp.dot(h1, wcat_ref[...], preferred_element_type=jnp.float32)
    ad2_ref[...] = jnp.dot(h1, wd_ref[...], preferred_element_type=jnp.float32)


def _layer1_combine(acc, den, b1, wcat, wd):
    n = acc.shape[1]
    return pl.pallas_call(
        _layer1_combine_body,
        grid=(n // _RB,),
        in_specs=[
            pl.BlockSpec((NC, _RB, 64), lambda i: (0, i, 0)),
            pl.BlockSpec((NC, NS, _RB, 4), lambda i: (0, 0, i, 0)),
            pl.BlockSpec((1, 128), lambda i: (0, 0)),
            pl.BlockSpec((128, 64), lambda i: (0, 0)),
            pl.BlockSpec((128, 1), lambda i: (0, 0)),
        ],
        out_specs=[
            pl.BlockSpec((_RB, 64), lambda i: (i, 0)),
            pl.BlockSpec((_RB, 1), lambda i: (i, 0)),
        ],
        out_shape=[
            jax.ShapeDtypeStruct((n, 64), jnp.float32),
            jax.ShapeDtypeStruct((n, 1), jnp.float32),
        ],
    )(acc, den, b1, wcat, wd)


def _final_body(acc_ref, den_ref, b2_ref, fcw_ref, fcb_ref, out_ref):
    dsum = den_ref[0, 0]
    for t in range(1, NS):
        dsum = dsum + den_ref[0, t]
    h2 = jnp.concatenate([acc_ref[0], acc_ref[1]], axis=1)
    h2 = h2 * (1.0 / (dsum + 1e-16)) + b2_ref[...]
    logits = jnp.dot(h2, fcw_ref[...], preferred_element_type=jnp.float32)
    logits = logits + fcb_ref[...]
    m = jnp.max(logits, axis=1, keepdims=True)
    lse = m + jnp.log(jnp.sum(jnp.exp(logits - m), axis=1, keepdims=True))
    out_ref[...] = logits - lse


def _final(acc, den, b2, fcw, fcb):
    n = acc.shape[1]
    return pl.pallas_call(
        _final_body,
        grid=(n // _RB,),
        in_specs=[
            pl.BlockSpec((NC, _RB, 16), lambda i: (0, i, 0)),
            pl.BlockSpec((1, NS, _RB, 1), lambda i: (0, 0, i, 0)),
            pl.BlockSpec((1, OUT), lambda i: (0, 0)),
            pl.BlockSpec((OUT, 2), lambda i: (0, 0)),
            pl.BlockSpec((1, 2), lambda i: (0, 0)),
        ],
        out_specs=pl.BlockSpec((_RB, 2), lambda i: (i, 0)),
        out_shape=jax.ShapeDtypeStruct((n, 2), jnp.float32),
    )(acc, den, b2, fcw, fcb)


def kernel(x, edge_index, W1, a_src1, a_dst1, b1, W2, a_src2, a_dst2, b2,
           fcW, fcb):
    x = x.astype(jnp.float32)
    loops = jnp.arange(N, dtype=jnp.int32)
    src = jnp.concatenate([edge_index[0].astype(jnp.int32), loops,
                           jnp.zeros((EPAD - EP,), jnp.int32)])
    dst = jnp.concatenate([edge_index[1].astype(jnp.int32), loops,
                           jnp.full((EPAD - EP,), N, jnp.int32)])

    eye = jnp.eye(HEADS, dtype=jnp.float32)
    A_src = (eye[:, :, None] * a_src1[:, None, :]).transpose(1, 2, 0)
    A_src = A_src.reshape(HEADS * HID, HEADS)
    A_dst = (eye[:, :, None] * a_dst1[:, None, :]).transpose(1, 2, 0)
    A_dst = A_dst.reshape(HEADS * HID, HEADS)
    WAs = W1 @ A_src
    z12 = jnp.zeros((D, 12), jnp.float32)
    W1cat = jnp.concatenate(
        [W1[:, :64], WAs[:, :4], z12, W1[:, 64:], WAs[:, 4:], z12],
        axis=1)
    W1d = W1 @ A_dst

    As2rep = jnp.tile(a_src2.T, (1, LANES))
    W2as = W2 @ As2rep
    W2cat = jnp.concatenate(
        [W2[:, :16], W2as, W2[:, 16:], W2as], axis=1)
    W2d = W2 @ a_dst2.T

    xpad = jnp.concatenate(
        [x, jnp.zeros((NPAD - N, D), jnp.float32)], axis=0)

    t1, ad1 = _feature_mm(xpad, W1cat, W1d)
    t1v = jnp.concatenate([t1[:, :80], t1[:, 80:]], axis=0)
    ad1v = jnp.concatenate([ad1[:, :4], ad1[:, 4:]], axis=0)
    acc1, den1 = _edge_pass_l1(t1v, ad1v.reshape(-1), src, dst)
    den1r = den1.reshape(NC, NS, NPAD, 4)

    t2, ad2 = _layer1_combine(acc1, den1r, b1.reshape(1, -1), W2cat, W2d)
    t2v = jnp.concatenate([t2[:, :32], t2[:, 32:]], axis=0)
    ad2v = jnp.concatenate([ad2, ad2], axis=0)
    acc2, den2 = _edge_pass_l2(t2v, ad2v.reshape(-1), src, dst)
    den2r = den2.reshape(NC, NS, NPAD, 1)

    out = _final(acc2, den2r, b2.reshape(1, -1), fcW, fcb.reshape(1, -1))
    return out[:N]
